# Initial kernel scaffold; baseline (speedup 1.0000x reference)
import jax
import jax.numpy as jnp
from jax.experimental import pallas as pl


def kernel(tokens, token_embed, W_pos):
    raise NotImplementedError("write your pallas kernel here")


# speedup vs baseline: 1.1758x; 1.1758x over previous
import jax
import jax.numpy as jnp
from jax.experimental import pallas as pl


def _pos_broadcast_kernel(w_ref, out_ref):
    out_ref[...] = jnp.broadcast_to(w_ref[...][None, :, :], out_ref.shape)


def kernel(tokens, token_embed, W_pos):
    batch, seq, d = token_embed.shape
    block_s = 512
    pos_embed = pl.pallas_call(
        _pos_broadcast_kernel,
        grid=(seq // block_s,),
        in_specs=[pl.BlockSpec((block_s, d), lambda j: (j, 0))],
        out_specs=pl.BlockSpec((batch, block_s, d), lambda j: (0, j, 0)),
        out_shape=jax.ShapeDtypeStruct((batch, seq, d), W_pos.dtype),
    )(W_pos)
    return (pos_embed, token_embed)


# speedup vs baseline: 1.1864x; 1.0090x over previous
import jax
import jax.numpy as jnp
from jax.experimental import pallas as pl


def _fused_kernel(w_ref, te_ref, pos_ref, te_out_ref):
    pos_ref[...] = jnp.broadcast_to(w_ref[...][None, :, :], pos_ref.shape)
    te_out_ref[...] = te_ref[...]


def kernel(tokens, token_embed, W_pos):
    batch, seq, d = token_embed.shape
    block_s = 256
    pos_embed, te_out = pl.pallas_call(
        _fused_kernel,
        grid=(seq // block_s,),
        in_specs=[
            pl.BlockSpec((block_s, d), lambda j: (j, 0)),
            pl.BlockSpec((batch, block_s, d), lambda j: (0, j, 0)),
        ],
        out_specs=[
            pl.BlockSpec((batch, block_s, d), lambda j: (0, j, 0)),
            pl.BlockSpec((batch, block_s, d), lambda j: (0, j, 0)),
        ],
        out_shape=[
            jax.ShapeDtypeStruct((batch, seq, d), W_pos.dtype),
            jax.ShapeDtypeStruct((batch, seq, d), token_embed.dtype),
        ],
    )(W_pos, token_embed)
    return (pos_embed, te_out)
